# baseline (speedup 1.0000x reference)
import functools

import jax
import jax.numpy as jnp
from jax.experimental import pallas as pl
from jax.experimental.pallas import tpu as pltpu

_N = 10000
_D = 128
_BM = 400


def _mlp1_kernel(x_in_ref, w1_ref, b1_ref, w2_ref, b2_ref, o_ref):
    h = jnp.maximum(
        jnp.dot(x_in_ref[...], w1_ref[...], preferred_element_type=jnp.float32)
        + b1_ref[...],
        0.0,
    )
    o_ref[...] = (
        jnp.dot(h, w2_ref[...], preferred_element_type=jnp.float32) + b2_ref[...]
    )


def _main_kernel(
    adj_ref,
    x_ref,
    m2w1_ref,
    m2b1_ref,
    m2w2_ref,
    m2b2_ref,
    f1u_w_ref,
    f1u_b_ref,
    f2u_w_ref,
    f2u_b_ref,
    f1r_w_ref,
    f1r_b_ref,
    f2r_w_ref,
    f2r_b_ref,
    f1_w_ref,
    f1_b_ref,
    f2_w_ref,
    f2_b_ref,
    o_ref,
):
    i = pl.program_id(0)
    x = x_ref[...]
    out = jnp.dot(
        adj_ref[...].astype(jnp.bfloat16),
        x.astype(jnp.bfloat16),
        preferred_element_type=jnp.float32,
    )
    h = jnp.maximum(
        jnp.dot(out, m2w1_ref[...], preferred_element_type=jnp.float32)
        + m2b1_ref[...],
        0.0,
    )
    out = jnp.dot(h, m2w2_ref[...], preferred_element_type=jnp.float32) + m2b2_ref[...]
    x_blk = x_ref[pl.ds(i * _BM, _BM), :]
    z = jax.nn.sigmoid(
        jnp.dot(out, f1u_w_ref[...], preferred_element_type=jnp.float32)
        + f1u_b_ref[...]
        + jnp.dot(x_blk, f2u_w_ref[...], preferred_element_type=jnp.float32)
        + f2u_b_ref[...]
    )
    r = jax.nn.sigmoid(
        jnp.dot(out, f1r_w_ref[...], preferred_element_type=jnp.float32)
        + f1r_b_ref[...]
        + jnp.dot(x_blk, f2r_w_ref[...], preferred_element_type=jnp.float32)
        + f2r_b_ref[...]
    )
    out2 = jnp.tanh(
        jnp.dot(out, f1_w_ref[...], preferred_element_type=jnp.float32)
        + f1_b_ref[...]
        + jnp.dot(r * x_blk, f2_w_ref[...], preferred_element_type=jnp.float32)
        + f2_b_ref[...]
    )
    o_ref[...] = (1.0 - z) * x_blk + z * out2


def _full(shape):
    return pl.BlockSpec(shape, lambda i: (0, 0))


@jax.jit
def kernel(x_in, adj, mlp1_W1, mlp1_b1, mlp1_W2, mlp1_b2, mlp2_W1, mlp2_b1,
           mlp2_W2, mlp2_b2, fc1u_W, fc1u_b, fc2u_W, fc2u_b, fc1r_W, fc1r_b,
           fc2r_W, fc2r_b, fc1_W, fc1_b, fc2_W, fc2_b):
    n, din = x_in.shape
    dout = mlp1_W2.shape[1]
    biases = [mlp1_b1, mlp1_b2, mlp2_b1, mlp2_b2, fc1u_b, fc2u_b, fc1r_b,
              fc2r_b, fc1_b, fc2_b]
    (mlp1_b1, mlp1_b2, mlp2_b1, mlp2_b2, fc1u_b, fc2u_b, fc1r_b, fc2r_b,
     fc1_b, fc2_b) = [b.reshape(1, -1) for b in biases]

    x = pl.pallas_call(
        _mlp1_kernel,
        out_shape=jax.ShapeDtypeStruct((n, dout), jnp.float32),
    )(x_in, mlp1_W1, mlp1_b1, mlp1_W2, mlp1_b2)

    grid = (n // _BM,)
    w_spec = _full((din, dout))
    b_spec = _full((1, dout))
    out = pl.pallas_call(
        _main_kernel,
        grid=grid,
        in_specs=[
            pl.BlockSpec((_BM, n), lambda i: (i, 0)),
            _full((n, dout)),
            w_spec, b_spec, w_spec, b_spec,
            w_spec, b_spec, w_spec, b_spec,
            w_spec, b_spec, w_spec, b_spec,
            w_spec, b_spec, w_spec, b_spec,
        ],
        out_specs=pl.BlockSpec((_BM, dout), lambda i: (i, 0)),
        out_shape=jax.ShapeDtypeStruct((n, dout), jnp.float32),
    )(adj, x, mlp2_W1, mlp2_b1, mlp2_W2, mlp2_b2, fc1u_W, fc1u_b, fc2u_W,
      fc2u_b, fc1r_W, fc1r_b, fc2r_W, fc2r_b, fc1_W, fc1_b, fc2_W, fc2_b)
    return out


# speedup vs baseline: 1.0381x; 1.0381x over previous
import jax
import jax.numpy as jnp
from jax.experimental import pallas as pl
from jax.experimental.pallas import tpu as pltpu

_BM = 400


def _fused_kernel(
    x_in_ref, adj_ref,
    m1w1_ref, m1b1_ref, m1w2_ref, m1b2_ref,
    m2w1_ref, m2b1_ref, m2w2_ref, m2b2_ref,
    f1u_w_ref, f1u_b_ref, f2u_w_ref, f2u_b_ref,
    f1r_w_ref, f1r_b_ref, f2r_w_ref, f2r_b_ref,
    f1_w_ref, f1_b_ref, f2_w_ref, f2_b_ref,
    o_ref,
    x_scr,
):
    m = pl.program_id(0)

    @pl.when(m == 0)
    def _compute_x():
        h = jnp.maximum(
            jnp.dot(x_in_ref[...], m1w1_ref[...],
                    preferred_element_type=jnp.float32) + m1b1_ref[...],
            0.0,
        )
        x_scr[...] = (
            jnp.dot(h, m1w2_ref[...], preferred_element_type=jnp.float32)
            + m1b2_ref[...]
        )

    out = jnp.dot(
        adj_ref[...].astype(jnp.bfloat16),
        x_scr[...].astype(jnp.bfloat16),
        preferred_element_type=jnp.float32,
    )
    h = jnp.maximum(
        jnp.dot(out, m2w1_ref[...], preferred_element_type=jnp.float32)
        + m2b1_ref[...],
        0.0,
    )
    out = jnp.dot(h, m2w2_ref[...], preferred_element_type=jnp.float32) + m2b2_ref[...]
    x_blk = x_scr[pl.ds(m * _BM, _BM), :]
    z = jax.nn.sigmoid(
        jnp.dot(out, f1u_w_ref[...], preferred_element_type=jnp.float32)
        + f1u_b_ref[...]
        + jnp.dot(x_blk, f2u_w_ref[...], preferred_element_type=jnp.float32)
        + f2u_b_ref[...]
    )
    r = jax.nn.sigmoid(
        jnp.dot(out, f1r_w_ref[...], preferred_element_type=jnp.float32)
        + f1r_b_ref[...]
        + jnp.dot(x_blk, f2r_w_ref[...], preferred_element_type=jnp.float32)
        + f2r_b_ref[...]
    )
    out2 = jnp.tanh(
        jnp.dot(out, f1_w_ref[...], preferred_element_type=jnp.float32)
        + f1_b_ref[...]
        + jnp.dot(r * x_blk, f2_w_ref[...], preferred_element_type=jnp.float32)
        + f2_b_ref[...]
    )
    o_ref[...] = (1.0 - z) * x_blk + z * out2


def _full(shape):
    return pl.BlockSpec(shape, lambda m: (0, 0))


@jax.jit
def kernel(x_in, adj, mlp1_W1, mlp1_b1, mlp1_W2, mlp1_b2, mlp2_W1, mlp2_b1,
           mlp2_W2, mlp2_b2, fc1u_W, fc1u_b, fc2u_W, fc2u_b, fc1r_W, fc1r_b,
           fc2r_W, fc2r_b, fc1_W, fc1_b, fc2_W, fc2_b):
    n, din = x_in.shape
    dout = mlp1_W2.shape[1]
    biases = [mlp1_b1, mlp1_b2, mlp2_b1, mlp2_b2, fc1u_b, fc2u_b, fc1r_b,
              fc2r_b, fc1_b, fc2_b]
    (mlp1_b1, mlp1_b2, mlp2_b1, mlp2_b2, fc1u_b, fc2u_b, fc1r_b, fc2r_b,
     fc1_b, fc2_b) = [b.reshape(1, -1) for b in biases]

    grid = (n // _BM,)
    w_spec = _full((din, dout))
    b_spec = _full((1, dout))
    out = pl.pallas_call(
        _fused_kernel,
        grid=grid,
        in_specs=[
            _full((n, din)),
            pl.BlockSpec((_BM, n), lambda m: (m, 0)),
            w_spec, b_spec, w_spec, b_spec,
            w_spec, b_spec, w_spec, b_spec,
            w_spec, b_spec, w_spec, b_spec,
            w_spec, b_spec, w_spec, b_spec,
            w_spec, b_spec, w_spec, b_spec,
        ],
        out_specs=pl.BlockSpec((_BM, dout), lambda m: (m, 0)),
        out_shape=jax.ShapeDtypeStruct((n, dout), jnp.float32),
        scratch_shapes=[
            pltpu.VMEM((n, dout), jnp.float32),
        ],
    )(x_in, adj, mlp1_W1, mlp1_b1, mlp1_W2, mlp1_b2, mlp2_W1, mlp2_b1,
      mlp2_W2, mlp2_b2, fc1u_W, fc1u_b, fc2u_W, fc2u_b, fc1r_W, fc1r_b,
      fc2r_W, fc2r_b, fc1_W, fc1_b, fc2_W, fc2_b)
    return out
